# baseline (speedup 1.0000x reference)
import functools

import jax
import jax.numpy as jnp
from jax import lax
from jax.experimental import pallas as pl
from jax.experimental.pallas import tpu as pltpu
from jax.experimental.pallas import tpu_sc as plsc

_B = 64
_ROI = 116
_SEG = _ROI
_SEG2 = 2 * _SEG
_D = 115
_F = 64
_N = _B * _SEG2
_EPG = 7424
_E = _B * _EPG
_NC = 2
_NS = 16
_NW = _NC * _NS
_EPT = _E // _NW
_CH = 128
_NCH = _EPT // _CH
_RPT = _N // _NS

_mesh = plsc.VectorSubcoreMesh(
    core_axis_name="c", subcore_axis_name="s", num_cores=_NC, num_subcores=_NS
)


def _mm_body(x_ref, wg_ref, wq_ref, h_ref, q_ref):
    xb = x_ref[...]
    h_ref[...] = jnp.dot(xb, wg_ref[...], preferred_element_type=jnp.float32)
    q_ref[...] = jnp.dot(xb, wq_ref[...], preferred_element_type=jnp.float32)


def _mlp_body(qa_ref, qb_ref, w1_ref, b1_ref, w2_ref, b2_ref, iw_ref):
    iw0 = qa_ref[...] + qb_ref[...]
    t = jnp.maximum(jnp.dot(iw0, w1_ref[...], preferred_element_type=jnp.float32)
                    + b1_ref[...], 0.0)
    iw_ref[...] = jnp.maximum(jnp.dot(t, w2_ref[...], preferred_element_type=jnp.float32)
                              + b2_ref[...], 0.0)


def _dinv_body(dp_ref, o_ref):
    dsum = dp_ref[0:1, :] + dp_ref[1:2, :] + 1.0
    o_ref[...] = lax.rsqrt(dsum)


def _ep_body(p0_ref, p1_ref, h_ref, dv_ref, bg_ref, g_ref):
    dv = dv_ref[...]
    g = p0_ref[...] + p1_ref[...] + dv * dv * h_ref[...] + bg_ref[...]
    g_ref[...] = jnp.maximum(g, 0.0)


def _fin_body(a_ref, w3_ref, b3_ref, w4_ref, b4_ref, o_ref):
    t = jnp.dot(a_ref[...], w3_ref[...], preferred_element_type=jnp.float32) + b3_ref[...]
    t = jnp.maximum(t, 0.0)
    o_ref[...] = jnp.dot(t, w4_ref[...], preferred_element_type=jnp.float32) + b4_ref[...]


def _deg_body(col_hbm, w_hbm, degp_hbm, col_v, w_v, wbuf, zbuf, deg_sh):
    c = lax.axis_index("c")
    s = lax.axis_index("s")
    tid = c * _NS + s

    pltpu.sync_copy(col_hbm.at[pl.ds(tid * _NCH, _NCH)], col_v)
    pltpu.sync_copy(w_hbm.at[pl.ds(tid * _NCH, _NCH)], w_v)

    zero16 = jnp.zeros((16,), jnp.float32)

    def _zb(i, _):
        zbuf[i, :] = zero16
        return 0
    lax.fori_loop(0, _RPT, _zb, 0)
    pltpu.sync_copy(zbuf, deg_sh.at[pl.ds(s * _RPT, _RPT)])
    plsc.subcore_barrier()

    def _chunk(k, _):
        def _fill(jj, _2):
            w16 = w_v[k, pl.ds(jj * 16, 16)]
            for u in range(16):
                e = jj * 16 + u
                wbuf[e, :] = jnp.full((16,), w16[u], jnp.float32)
            return 0
        lax.fori_loop(0, _CH // 16, _fill, 0)
        pltpu.sync_copy(wbuf, deg_sh.at[col_v.at[k]], add=True)
        return 0
    lax.fori_loop(0, _NCH, _chunk, 0)

    plsc.subcore_barrier()
    pltpu.sync_copy(deg_sh.at[pl.ds(s * _RPT, _RPT)],
                    degp_hbm.at[c, pl.ds(s * _RPT, _RPT)])


_deg_call = pl.kernel(
    _deg_body,
    out_type=jax.ShapeDtypeStruct((_NC, _N, 16), jnp.float32),
    mesh=_mesh,
    compiler_params=pltpu.CompilerParams(use_tc_tiling_on_sc=False, needs_layout_passes=False),
    scratch_types=[
        pltpu.VMEM((_NCH, _CH), jnp.int32),
        pltpu.VMEM((_NCH, _CH), jnp.float32),
        pltpu.VMEM((_CH, 16), jnp.float32),
        pltpu.VMEM((_RPT, 16), jnp.float32),
        pltpu.VMEM_SHARED((_N, 16), jnp.float32),
    ],
)


def _agg_body(row_hbm, col_hbm, w_hbm, dinv_hbm, h_hbm, z_hbm, outp_hbm,
              row_v, col_v, w_v, dinv_v, hbuf0, out_sh, sg0):
    c = lax.axis_index("c")
    s = lax.axis_index("s")
    tid = c * _NS + s

    pltpu.sync_copy(dinv_hbm, dinv_v)
    pltpu.sync_copy(row_hbm.at[pl.ds(tid * _NCH, _NCH)], row_v)
    pltpu.sync_copy(col_hbm.at[pl.ds(tid * _NCH, _NCH)], col_v)
    pltpu.sync_copy(w_hbm.at[pl.ds(tid * _NCH, _NCH)], w_v)
    pltpu.sync_copy(z_hbm.at[pl.ds(s * _RPT, _RPT)],
                    out_sh.at[pl.ds(s * _RPT, _RPT)])

    def _normk(k, _):
        for j in range(_CH // 16):
            sl = pl.ds(j * 16, 16)
            dr = plsc.load_gather(dinv_v, [row_v[k, sl]])
            dc = plsc.load_gather(dinv_v, [col_v[k, sl]])
            w_v[k, sl] = dr * w_v[k, sl] * dc
        return 0
    lax.fori_loop(0, _NCH, _normk, 0)
    plsc.subcore_barrier()

    def _scale(k, buf):
        def _sb(jj, _2):
            n16 = w_v[k, pl.ds(jj * 16, 16)]
            for u in range(16):
                e = jj * 16 + u
                sc = n16[u]
                for f in range(_F // 16):
                    buf[e, pl.ds(f * 16, 16)] = buf[e, pl.ds(f * 16, 16)] * sc
            return 0
        lax.fori_loop(0, _CH // 16, _sb, 0)

    def _chunk(k, _):
        pltpu.async_copy(h_hbm.at[row_v.at[k]], hbuf0, sg0).wait()
        _scale(k, hbuf0)
        pltpu.sync_copy(hbuf0, out_sh.at[col_v.at[k]], add=True)
        return 0
    lax.fori_loop(0, _NCH, _chunk, 0)

    plsc.subcore_barrier()
    pltpu.sync_copy(out_sh.at[pl.ds(s * _RPT, _RPT)],
                    outp_hbm.at[c, pl.ds(s * _RPT, _RPT)])


_agg_call = pl.kernel(
    _agg_body,
    out_type=jax.ShapeDtypeStruct((_NC, _N, _F), jnp.float32),
    mesh=_mesh,
    compiler_params=pltpu.CompilerParams(use_tc_tiling_on_sc=False, needs_layout_passes=False),
    scratch_types=[
        pltpu.VMEM((_NCH, _CH), jnp.int32),
        pltpu.VMEM((_NCH, _CH), jnp.int32),
        pltpu.VMEM((_NCH, _CH), jnp.float32),
        pltpu.VMEM((_N,), jnp.float32),
        pltpu.VMEM((_CH, _F), jnp.float32),
        pltpu.VMEM_SHARED((_N, _F), jnp.float32),
        pltpu.SemaphoreType.DMA,
    ],
)


def kernel(x, edge_index, edge_weight, batch, device, conv1_w, conv1_b,
           W1, b1, W2, b2, Wg, bg, W3, b3, W4, b4):
    f32 = jnp.float32
    x = x.astype(f32)

    wq = jnp.stack([conv1_w[:_D], conv1_w[_D:2 * _D]], axis=1)
    nrb = 8
    rb = _N // nrb
    h, q = pl.pallas_call(
        _mm_body,
        grid=(nrb,),
        in_specs=[
            pl.BlockSpec((rb, _D), lambda i: (i, 0)),
            pl.BlockSpec((_D, _F), lambda i: (0, 0)),
            pl.BlockSpec((_D, 2), lambda i: (0, 0)),
        ],
        out_specs=[
            pl.BlockSpec((rb, _F), lambda i: (i, 0)),
            pl.BlockSpec((rb, 2), lambda i: (i, 0)),
        ],
        out_shape=[
            jax.ShapeDtypeStruct((_N, _F), f32),
            jax.ShapeDtypeStruct((_N, 2), f32),
        ],
    )(x, Wg, wq)

    qa = q[:, 0].reshape(_B, 2, _SEG)[:, 0]
    qb = q[:, 1].reshape(_B, 2, _SEG)[:, 1]
    b1e = (b1 + conv1_b[0] * jnp.sum(W1, axis=0)).reshape(1, -1)
    iw = pl.pallas_call(
        _mlp_body,
        in_specs=[pl.BlockSpec(qa.shape, lambda: (0, 0)),
                  pl.BlockSpec(qb.shape, lambda: (0, 0)),
                  pl.BlockSpec(W1.shape, lambda: (0, 0)),
                  pl.BlockSpec(b1e.shape, lambda: (0, 0)),
                  pl.BlockSpec(W2.shape, lambda: (0, 0)),
                  pl.BlockSpec((1, _ROI), lambda: (0, 0))],
        out_specs=pl.BlockSpec((_B, _ROI), lambda: (0, 0)),
        out_shape=jax.ShapeDtypeStruct((_B, _ROI), f32),
    )(qa, qb, W1, b1e, W2, b2.reshape(1, -1))

    ew = edge_weight.reshape(_B, _EPG).at[:, _EPG - _SEG:].set(iw).reshape(-1)

    row2d = edge_index[0].reshape(_E // _CH, _CH).astype(jnp.int32)
    col2d = edge_index[1].reshape(_E // _CH, _CH).astype(jnp.int32)
    ew2d = ew.reshape(_E // _CH, _CH)

    degp = _deg_call(col2d, ew2d)

    dp = degp[:, :, 0].reshape(_NC, _N)
    dinv = pl.pallas_call(
        _dinv_body,
        in_specs=[pl.BlockSpec((_NC, _N), lambda: (0, 0))],
        out_specs=pl.BlockSpec((1, _N), lambda: (0, 0)),
        out_shape=jax.ShapeDtypeStruct((1, _N), f32),
    )(dp).reshape(_N)

    zeros = jnp.zeros((_N, _F), f32)
    outp = _agg_call(row2d, col2d, ew2d, dinv, h, zeros)

    g = pl.pallas_call(
        _ep_body,
        grid=(nrb,),
        in_specs=[
            pl.BlockSpec((rb, _F), lambda i: (i, 0)),
            pl.BlockSpec((rb, _F), lambda i: (i, 0)),
            pl.BlockSpec((rb, _F), lambda i: (i, 0)),
            pl.BlockSpec((rb, 1), lambda i: (i, 0)),
            pl.BlockSpec((1, _F), lambda i: (0, 0)),
        ],
        out_specs=pl.BlockSpec((rb, _F), lambda i: (i, 0)),
        out_shape=jax.ShapeDtypeStruct((_N, _F), f32),
    )(outp[0], outp[1], h, dinv.reshape(_N, 1), bg.reshape(1, _F))

    flat = g.reshape(_B, _SEG2 * _F)
    out = pl.pallas_call(
        _fin_body,
        in_specs=[pl.BlockSpec(flat.shape, lambda: (0, 0)),
                  pl.BlockSpec(W3.shape, lambda: (0, 0)),
                  pl.BlockSpec((1, W3.shape[1]), lambda: (0, 0)),
                  pl.BlockSpec(W4.shape, lambda: (0, 0)),
                  pl.BlockSpec((1, 1), lambda: (0, 0))],
        out_specs=pl.BlockSpec((_B, 1), lambda: (0, 0)),
        out_shape=jax.ShapeDtypeStruct((_B, 1), f32),
    )(flat, W3, b3.reshape(1, -1), W4, b4.reshape(1, 1))

    return out, iw


# speedup vs baseline: 1.3724x; 1.3724x over previous
import jax
import jax.numpy as jnp
from jax import lax
from jax.experimental import pallas as pl
from jax.experimental.pallas import tpu as pltpu
from jax.experimental.pallas import tpu_sc as plsc

_B = 64
_ROI = 116
_SEG = _ROI
_SEG2 = 2 * _SEG
_D = 115
_F = 64
_N = _B * _SEG2
_EPG = 7424
_E = _B * _EPG
_NC = 2
_NS = 16
_NW = _NC * _NS
_EPT = _E // _NW
_CH = 128
_NCH = _EPT // _CH
_RPT = _N // _NS

_mesh = plsc.VectorSubcoreMesh(
    core_axis_name="c", subcore_axis_name="s", num_cores=_NC, num_subcores=_NS
)
_sc_params = pltpu.CompilerParams(
    use_tc_tiling_on_sc=False, needs_layout_passes=False
)


def _mm_body(x_ref, wg_ref, wq_ref, h_ref, q_ref):
    xb = x_ref[...]
    h_ref[...] = jnp.dot(xb, wg_ref[...], preferred_element_type=jnp.float32)
    q_ref[...] = jnp.dot(xb, wq_ref[...], preferred_element_type=jnp.float32)


def _mlp_body(qa_ref, qb_ref, w1_ref, b1_ref, w2_ref, b2_ref, iw_ref):
    iw0 = qa_ref[...] + qb_ref[...]
    t = jnp.maximum(jnp.dot(iw0, w1_ref[...], preferred_element_type=jnp.float32)
                    + b1_ref[...], 0.0)
    iw_ref[...] = jnp.maximum(jnp.dot(t, w2_ref[...], preferred_element_type=jnp.float32)
                              + b2_ref[...], 0.0)


def _dinv_body(dp_ref, o_ref):
    dsum = dp_ref[0:1, :] + dp_ref[1:2, :] + 1.0
    o_ref[...] = lax.rsqrt(dsum)


def _ep_body(p0_ref, p1_ref, h_ref, dv_ref, bg_ref, g_ref):
    dv = dv_ref[...]
    g = p0_ref[...] + p1_ref[...] + dv * dv * h_ref[...] + bg_ref[...]
    g_ref[...] = jnp.maximum(g, 0.0)


def _fin_body(a_ref, w3_ref, b3_ref, w4_ref, b4_ref, o_ref):
    t = jnp.dot(a_ref[...], w3_ref[...], preferred_element_type=jnp.float32) + b3_ref[...]
    t = jnp.maximum(t, 0.0)
    o_ref[...] = jnp.dot(t, w4_ref[...], preferred_element_type=jnp.float32) + b4_ref[...]


def _deg_body(col_hbm, w_hbm, degp_hbm, col_v, w_v, wbuf, zbuf, deg_sh):
    c = lax.axis_index("c")
    s = lax.axis_index("s")
    tid = c * _NS + s

    pltpu.sync_copy(col_hbm.at[pl.ds(tid * _NCH, _NCH)], col_v)
    pltpu.sync_copy(w_hbm.at[pl.ds(tid * _NCH, _NCH)], w_v)

    zero16 = jnp.zeros((16,), jnp.float32)

    def _zb(i, _):
        zbuf[i, :] = zero16
        return 0
    lax.fori_loop(0, _RPT, _zb, 0)
    pltpu.sync_copy(zbuf, deg_sh.at[pl.ds(s * _RPT, _RPT)])
    plsc.subcore_barrier()

    def _chunk(k, _):
        def _fill(jj, _2):
            w16 = w_v[k, pl.ds(jj * 16, 16)]
            for u in range(16):
                e = jj * 16 + u
                wbuf[e, :] = jnp.full((16,), w16[u], jnp.float32)
            return 0
        lax.fori_loop(0, _CH // 16, _fill, 0)
        pltpu.sync_copy(wbuf, deg_sh.at[col_v.at[k]], add=True)
        return 0
    lax.fori_loop(0, _NCH, _chunk, 0)

    plsc.subcore_barrier()
    pltpu.sync_copy(deg_sh.at[pl.ds(s * _RPT, _RPT)],
                    degp_hbm.at[c, pl.ds(s * _RPT, _RPT)])


_deg_call = pl.kernel(
    _deg_body,
    out_type=jax.ShapeDtypeStruct((_NC, _N, 16), jnp.float32),
    mesh=_mesh,
    compiler_params=_sc_params,
    scratch_types=[
        pltpu.VMEM((_NCH, _CH), jnp.int32),
        pltpu.VMEM((_NCH, _CH), jnp.float32),
        pltpu.VMEM((_CH, 16), jnp.float32),
        pltpu.VMEM((_RPT, 16), jnp.float32),
        pltpu.VMEM_SHARED((_N, 16), jnp.float32),
    ],
)


def _agg_body(row_hbm, col_hbm, w_hbm, dinv_hbm, h_hbm, z_hbm, outp_hbm,
              row_v, col_v, w_v, dinv_v, norm_v, hbuf, out_sh, sem):
    c = lax.axis_index("c")
    s = lax.axis_index("s")
    tid = c * _NS + s

    pltpu.sync_copy(dinv_hbm, dinv_v)
    pltpu.sync_copy(row_hbm.at[pl.ds(tid * _NCH, _NCH)], row_v)
    pltpu.sync_copy(col_hbm.at[pl.ds(tid * _NCH, _NCH)], col_v)
    pltpu.sync_copy(w_hbm.at[pl.ds(tid * _NCH, _NCH)], w_v)
    pltpu.sync_copy(z_hbm.at[pl.ds(s * _RPT, _RPT)],
                    out_sh.at[pl.ds(s * _RPT, _RPT)])
    plsc.subcore_barrier()

    def _chunk(k, _):
        pltpu.async_copy(h_hbm.at[row_v.at[k]], hbuf, sem).wait()
        for j in range(_CH // 16):
            sl = pl.ds(j * 16, 16)
            dr = plsc.load_gather(dinv_v, [row_v[k, sl]])
            dc = plsc.load_gather(dinv_v, [col_v[k, sl]])
            norm_v[sl] = dr * w_v[k, sl] * dc

        def _sb(jj, _2):
            n16 = norm_v[pl.ds(jj * 16, 16)]
            for u in range(16):
                e = jj * 16 + u
                sc = n16[u]
                for f in range(_F // 16):
                    hbuf[e, pl.ds(f * 16, 16)] = hbuf[e, pl.ds(f * 16, 16)] * sc
            return 0
        lax.fori_loop(0, _CH // 16, _sb, 0)
        pltpu.sync_copy(hbuf, out_sh.at[col_v.at[k]], add=True)
        return 0
    lax.fori_loop(0, _NCH, _chunk, 0)

    plsc.subcore_barrier()
    pltpu.sync_copy(out_sh.at[pl.ds(s * _RPT, _RPT)],
                    outp_hbm.at[c, pl.ds(s * _RPT, _RPT)])


_agg_call = pl.kernel(
    _agg_body,
    out_type=jax.ShapeDtypeStruct((_NC, _N, _F), jnp.float32),
    mesh=_mesh,
    compiler_params=_sc_params,
    scratch_types=[
        pltpu.VMEM((_NCH, _CH), jnp.int32),
        pltpu.VMEM((_NCH, _CH), jnp.int32),
        pltpu.VMEM((_NCH, _CH), jnp.float32),
        pltpu.VMEM((_N,), jnp.float32),
        pltpu.VMEM((_CH,), jnp.float32),
        pltpu.VMEM((_CH, _F), jnp.float32),
        pltpu.VMEM_SHARED((_N, _F), jnp.float32),
        pltpu.SemaphoreType.DMA,
    ],
)


def kernel(x, edge_index, edge_weight, batch, device, conv1_w, conv1_b,
           W1, b1, W2, b2, Wg, bg, W3, b3, W4, b4):
    f32 = jnp.float32
    x = x.astype(f32)

    wq = jnp.stack([conv1_w[:_D], conv1_w[_D:2 * _D]], axis=1)
    nrb = 8
    rb = _N // nrb
    h, q = pl.pallas_call(
        _mm_body,
        grid=(nrb,),
        in_specs=[
            pl.BlockSpec((rb, _D), lambda i: (i, 0)),
            pl.BlockSpec((_D, _F), lambda i: (0, 0)),
            pl.BlockSpec((_D, 2), lambda i: (0, 0)),
        ],
        out_specs=[
            pl.BlockSpec((rb, _F), lambda i: (i, 0)),
            pl.BlockSpec((rb, 2), lambda i: (i, 0)),
        ],
        out_shape=[
            jax.ShapeDtypeStruct((_N, _F), f32),
            jax.ShapeDtypeStruct((_N, 2), f32),
        ],
    )(x, Wg, wq)

    qa = q[:, 0].reshape(_B, 2, _SEG)[:, 0]
    qb = q[:, 1].reshape(_B, 2, _SEG)[:, 1]
    b1e = (b1 + conv1_b[0] * jnp.sum(W1, axis=0)).reshape(1, -1)
    iw = pl.pallas_call(
        _mlp_body,
        in_specs=[pl.BlockSpec(qa.shape, lambda: (0, 0)),
                  pl.BlockSpec(qb.shape, lambda: (0, 0)),
                  pl.BlockSpec(W1.shape, lambda: (0, 0)),
                  pl.BlockSpec(b1e.shape, lambda: (0, 0)),
                  pl.BlockSpec(W2.shape, lambda: (0, 0)),
                  pl.BlockSpec((1, _ROI), lambda: (0, 0))],
        out_specs=pl.BlockSpec((_B, _ROI), lambda: (0, 0)),
        out_shape=jax.ShapeDtypeStruct((_B, _ROI), f32),
    )(qa, qb, W1, b1e, W2, b2.reshape(1, -1))

    ew = edge_weight.reshape(_B, _EPG).at[:, _EPG - _SEG:].set(iw).reshape(-1)

    row2d = edge_index[0].reshape(_E // _CH, _CH).astype(jnp.int32)
    col2d = edge_index[1].reshape(_E // _CH, _CH).astype(jnp.int32)
    ew2d = ew.reshape(_E // _CH, _CH)

    degp = _deg_call(col2d, ew2d)

    dp = degp[:, :, 0].reshape(_NC, _N)
    dinv = pl.pallas_call(
        _dinv_body,
        in_specs=[pl.BlockSpec((_NC, _N), lambda: (0, 0))],
        out_specs=pl.BlockSpec((1, _N), lambda: (0, 0)),
        out_shape=jax.ShapeDtypeStruct((1, _N), f32),
    )(dp).reshape(_N)

    zeros = jnp.zeros((_N, _F), f32)
    outp = _agg_call(row2d, col2d, ew2d, dinv, h, zeros)

    g = pl.pallas_call(
        _ep_body,
        grid=(nrb,),
        in_specs=[
            pl.BlockSpec((rb, _F), lambda i: (i, 0)),
            pl.BlockSpec((rb, _F), lambda i: (i, 0)),
            pl.BlockSpec((rb, _F), lambda i: (i, 0)),
            pl.BlockSpec((rb, 1), lambda i: (i, 0)),
            pl.BlockSpec((1, _F), lambda i: (0, 0)),
        ],
        out_specs=pl.BlockSpec((rb, _F), lambda i: (i, 0)),
        out_shape=jax.ShapeDtypeStruct((_N, _F), f32),
    )(outp[0], outp[1], h, dinv.reshape(_N, 1), bg.reshape(1, _F))

    flat = g.reshape(_B, _SEG2 * _F)
    out = pl.pallas_call(
        _fin_body,
        in_specs=[pl.BlockSpec(flat.shape, lambda: (0, 0)),
                  pl.BlockSpec(W3.shape, lambda: (0, 0)),
                  pl.BlockSpec((1, W3.shape[1]), lambda: (0, 0)),
                  pl.BlockSpec(W4.shape, lambda: (0, 0)),
                  pl.BlockSpec((1, 1), lambda: (0, 0))],
        out_specs=pl.BlockSpec((_B, 1), lambda: (0, 0)),
        out_shape=jax.ShapeDtypeStruct((_B, 1), f32),
    )(flat, W3, b3.reshape(1, -1), W4, b4.reshape(1, 1))

    return out, iw


# speedup vs baseline: 1.4146x; 1.0308x over previous
import jax
import jax.numpy as jnp
from jax import lax
from jax.experimental import pallas as pl
from jax.experimental.pallas import tpu as pltpu
from jax.experimental.pallas import tpu_sc as plsc

_B = 64
_ROI = 116
_SEG = _ROI
_SEG2 = 2 * _SEG
_D = 115
_F = 64
_N = _B * _SEG2
_EPG = 7424
_E = _B * _EPG
_NC = 2
_NS = 16
_NW = _NC * _NS
_EPT = _E // _NW
_CH = 128
_NCH = _EPT // _CH
_RPT = _N // _NS

_mesh = plsc.VectorSubcoreMesh(
    core_axis_name="c", subcore_axis_name="s", num_cores=_NC, num_subcores=_NS
)
_sc_params = pltpu.CompilerParams(
    use_tc_tiling_on_sc=False, needs_layout_passes=False
)


def _h_body(x_ref, wg_ref, h_ref):
    h_ref[...] = jnp.dot(x_ref[...], wg_ref[...], preferred_element_type=jnp.float32)


def _q_body(x_ref, wq_ref, q_ref):
    q_ref[...] = jnp.dot(x_ref[...], wq_ref[...], preferred_element_type=jnp.float32)


def _mlp_body(qa_ref, qb_ref, w1_ref, b1_ref, w2_ref, b2_ref, iw_ref):
    iw0 = qa_ref[...] + qb_ref[...]
    t = jnp.maximum(jnp.dot(iw0, w1_ref[...], preferred_element_type=jnp.float32)
                    + b1_ref[...], 0.0)
    iw_ref[...] = jnp.maximum(jnp.dot(t, w2_ref[...], preferred_element_type=jnp.float32)
                              + b2_ref[...], 0.0)


def _dinv_body(dp_ref, o_ref):
    dsum = dp_ref[0:1, :] + dp_ref[1:2, :] + 1.0
    o_ref[...] = lax.rsqrt(dsum)


def _ep_body(p0_ref, p1_ref, h_ref, dv_ref, bg_ref, g_ref):
    dv = dv_ref[...]
    g = p0_ref[...] + p1_ref[...] + dv * dv * h_ref[...] + bg_ref[...]
    g_ref[...] = jnp.maximum(g, 0.0)


def _fin_body(a_ref, w3_ref, b3_ref, w4_ref, b4_ref, o_ref):
    t = jnp.dot(a_ref[...], w3_ref[...], preferred_element_type=jnp.float32) + b3_ref[...]
    t = jnp.maximum(t, 0.0)
    o_ref[...] = jnp.dot(t, w4_ref[...], preferred_element_type=jnp.float32) + b4_ref[...]


def _deg_body(col_hbm, w_hbm, degp_hbm, col_v, w_v, wbuf, zbuf, deg_sh):
    c = lax.axis_index("c")
    s = lax.axis_index("s")
    tid = c * _NS + s

    pltpu.sync_copy(col_hbm.at[pl.ds(tid * _NCH, _NCH)], col_v)
    pltpu.sync_copy(w_hbm.at[pl.ds(tid * _NCH, _NCH)], w_v)

    zero16 = jnp.zeros((16,), jnp.float32)

    def _zb(i, _):
        zbuf[i, :] = zero16
        return 0
    lax.fori_loop(0, _RPT, _zb, 0)
    pltpu.sync_copy(zbuf, deg_sh.at[pl.ds(s * _RPT, _RPT)])
    plsc.subcore_barrier()

    iota16 = lax.iota(jnp.int32, 16)
    zero_i16 = jnp.zeros((16,), jnp.int32)

    def _chunk(k, _):
        def _fill(jj, _2):
            w16 = w_v[k, pl.ds(jj * 16, 16)]
            plsc.store_scatter(wbuf, [jj * 16 + iota16, zero_i16], w16)
            return 0
        lax.fori_loop(0, _CH // 16, _fill, 0)
        pltpu.sync_copy(wbuf, deg_sh.at[col_v.at[k]], add=True)
        return 0
    lax.fori_loop(0, _NCH, _chunk, 0)

    plsc.subcore_barrier()
    pltpu.sync_copy(deg_sh.at[pl.ds(s * _RPT, _RPT)],
                    degp_hbm.at[c, pl.ds(s * _RPT, _RPT)])


_deg_call = pl.kernel(
    _deg_body,
    out_type=jax.ShapeDtypeStruct((_NC, _N, 16), jnp.float32),
    mesh=_mesh,
    compiler_params=_sc_params,
    scratch_types=[
        pltpu.VMEM((_NCH, _CH), jnp.int32),
        pltpu.VMEM((_NCH, _CH), jnp.float32),
        pltpu.VMEM((_CH, 16), jnp.float32),
        pltpu.VMEM((_RPT, 16), jnp.float32),
        pltpu.VMEM_SHARED((_N, 16), jnp.float32),
    ],
)


def _agg_body(row_hbm, col_hbm, w_hbm, dinv_hbm, h_hbm, z_hbm, outp_hbm,
              row_v, col_v, w_v, dinv_v, norm_v, hbuf, out_sh, sem):
    c = lax.axis_index("c")
    s = lax.axis_index("s")
    tid = c * _NS + s

    pltpu.sync_copy(dinv_hbm, dinv_v)
    pltpu.sync_copy(row_hbm.at[pl.ds(tid * _NCH, _NCH)], row_v)
    pltpu.sync_copy(col_hbm.at[pl.ds(tid * _NCH, _NCH)], col_v)
    pltpu.sync_copy(w_hbm.at[pl.ds(tid * _NCH, _NCH)], w_v)
    pltpu.sync_copy(z_hbm.at[pl.ds(s * _RPT, _RPT)],
                    out_sh.at[pl.ds(s * _RPT, _RPT)])
    plsc.subcore_barrier()

    def _chunk(k, _):
        pltpu.async_copy(h_hbm.at[row_v.at[k]], hbuf, sem).wait()
        for j in range(_CH // 16):
            sl = pl.ds(j * 16, 16)
            dr = plsc.load_gather(dinv_v, [row_v[k, sl]])
            dc = plsc.load_gather(dinv_v, [col_v[k, sl]])
            norm_v[sl] = dr * w_v[k, sl] * dc

        def _sb(jj, _2):
            n16 = norm_v[pl.ds(jj * 16, 16)]
            for u in range(16):
                e = jj * 16 + u
                sc = n16[u]
                for f in range(_F // 16):
                    hbuf[e, pl.ds(f * 16, 16)] = hbuf[e, pl.ds(f * 16, 16)] * sc
            return 0
        lax.fori_loop(0, _CH // 16, _sb, 0)
        pltpu.sync_copy(hbuf, out_sh.at[col_v.at[k]], add=True)
        return 0
    lax.fori_loop(0, _NCH, _chunk, 0)

    plsc.subcore_barrier()
    pltpu.sync_copy(out_sh.at[pl.ds(s * _RPT, _RPT)],
                    outp_hbm.at[c, pl.ds(s * _RPT, _RPT)])


_agg_call = pl.kernel(
    _agg_body,
    out_type=jax.ShapeDtypeStruct((_NC, _N, _F), jnp.float32),
    mesh=_mesh,
    compiler_params=_sc_params,
    scratch_types=[
        pltpu.VMEM((_NCH, _CH), jnp.int32),
        pltpu.VMEM((_NCH, _CH), jnp.int32),
        pltpu.VMEM((_NCH, _CH), jnp.float32),
        pltpu.VMEM((_N,), jnp.float32),
        pltpu.VMEM((_CH,), jnp.float32),
        pltpu.VMEM((_CH, _F), jnp.float32),
        pltpu.VMEM_SHARED((_N, _F), jnp.float32),
        pltpu.SemaphoreType.DMA,
    ],
)


def kernel(x, edge_index, edge_weight, batch, device, conv1_w, conv1_b,
           W1, b1, W2, b2, Wg, bg, W3, b3, W4, b4):
    f32 = jnp.float32
    x = x.astype(f32)

    wq = jnp.stack([conv1_w[:_D], conv1_w[_D:2 * _D]], axis=1)
    nrb = 8
    rb = _N // nrb
    q = pl.pallas_call(
        _q_body,
        grid=(nrb,),
        in_specs=[
            pl.BlockSpec((rb, _D), lambda i: (i, 0)),
            pl.BlockSpec((_D, 2), lambda i: (0, 0)),
        ],
        out_specs=pl.BlockSpec((rb, 2), lambda i: (i, 0)),
        out_shape=jax.ShapeDtypeStruct((_N, 2), f32),
    )(x, wq)

    qa = q[:, 0].reshape(_B, 2, _SEG)[:, 0]
    qb = q[:, 1].reshape(_B, 2, _SEG)[:, 1]
    b1e = (b1 + conv1_b[0] * jnp.sum(W1, axis=0)).reshape(1, -1)
    iw = pl.pallas_call(
        _mlp_body,
        in_specs=[pl.BlockSpec(qa.shape, lambda: (0, 0)),
                  pl.BlockSpec(qb.shape, lambda: (0, 0)),
                  pl.BlockSpec(W1.shape, lambda: (0, 0)),
                  pl.BlockSpec(b1e.shape, lambda: (0, 0)),
                  pl.BlockSpec(W2.shape, lambda: (0, 0)),
                  pl.BlockSpec((1, _ROI), lambda: (0, 0))],
        out_specs=pl.BlockSpec((_B, _ROI), lambda: (0, 0)),
        out_shape=jax.ShapeDtypeStruct((_B, _ROI), f32),
    )(qa, qb, W1, b1e, W2, b2.reshape(1, -1))

    ew = edge_weight.reshape(_B, _EPG).at[:, _EPG - _SEG:].set(iw).reshape(-1)

    row2d = edge_index[0].reshape(_E // _CH, _CH).astype(jnp.int32)
    col2d = edge_index[1].reshape(_E // _CH, _CH).astype(jnp.int32)
    ew2d = ew.reshape(_E // _CH, _CH)

    degp = _deg_call(col2d, ew2d)

    h = pl.pallas_call(
        _h_body,
        grid=(nrb,),
        in_specs=[
            pl.BlockSpec((rb, _D), lambda i: (i, 0)),
            pl.BlockSpec((_D, _F), lambda i: (0, 0)),
        ],
        out_specs=pl.BlockSpec((rb, _F), lambda i: (i, 0)),
        out_shape=jax.ShapeDtypeStruct((_N, _F), f32),
    )(x, Wg)

    dp = degp[:, :, 0].reshape(_NC, _N)
    dinv = pl.pallas_call(
        _dinv_body,
        in_specs=[pl.BlockSpec((_NC, _N), lambda: (0, 0))],
        out_specs=pl.BlockSpec((1, _N), lambda: (0, 0)),
        out_shape=jax.ShapeDtypeStruct((1, _N), f32),
    )(dp).reshape(_N)

    zeros = jnp.zeros((_N, _F), f32)
    outp = _agg_call(row2d, col2d, ew2d, dinv, h, zeros)

    g = pl.pallas_call(
        _ep_body,
        grid=(nrb,),
        in_specs=[
            pl.BlockSpec((rb, _F), lambda i: (i, 0)),
            pl.BlockSpec((rb, _F), lambda i: (i, 0)),
            pl.BlockSpec((rb, _F), lambda i: (i, 0)),
            pl.BlockSpec((rb, 1), lambda i: (i, 0)),
            pl.BlockSpec((1, _F), lambda i: (0, 0)),
        ],
        out_specs=pl.BlockSpec((rb, _F), lambda i: (i, 0)),
        out_shape=jax.ShapeDtypeStruct((_N, _F), f32),
    )(outp[0], outp[1], h, dinv.reshape(_N, 1), bg.reshape(1, _F))

    flat = g.reshape(_B, _SEG2 * _F)
    out = pl.pallas_call(
        _fin_body,
        in_specs=[pl.BlockSpec(flat.shape, lambda: (0, 0)),
                  pl.BlockSpec(W3.shape, lambda: (0, 0)),
                  pl.BlockSpec((1, W3.shape[1]), lambda: (0, 0)),
                  pl.BlockSpec(W4.shape, lambda: (0, 0)),
                  pl.BlockSpec((1, 1), lambda: (0, 0))],
        out_specs=pl.BlockSpec((_B, 1), lambda: (0, 0)),
        out_shape=jax.ShapeDtypeStruct((_B, 1), f32),
    )(flat, W3, b3.reshape(1, -1), W4, b4.reshape(1, 1))

    return out, iw
